# Initial kernel scaffold; baseline (speedup 1.0000x reference)
import jax
import jax.numpy as jnp
from jax.experimental import pallas as pl


def kernel(x, adj1, adj2, W, b):
    raise NotImplementedError("write your pallas kernel here")


# speedup vs baseline: 1.0085x; 1.0085x over previous
import functools

import jax
import jax.numpy as jnp
from jax.experimental import pallas as pl
from jax.experimental.pallas import tpu as pltpu


def _fused_block(adj1_ref, adj2_ref, x1_ref, x2_ref, w1_ref, w2_ref, b_ref,
                 out_ref):
    a1 = adj1_ref[...].astype(jnp.bfloat16)
    a2 = adj2_ref[...].astype(jnp.bfloat16)
    h1 = jax.lax.dot_general(
        a1, x1_ref[...], (((1,), (0,)), ((), ())),
        preferred_element_type=jnp.float32)
    h2 = jax.lax.dot_general(
        a2, x2_ref[...], (((1,), (0,)), ((), ())),
        preferred_element_type=jnp.float32)
    o = jax.lax.dot_general(
        h1, w1_ref[...], (((1,), (0,)), ((), ())),
        preferred_element_type=jnp.float32)
    o += jax.lax.dot_general(
        h2, w2_ref[...], (((1,), (0,)), ((), ())),
        preferred_element_type=jnp.float32)
    out_ref[...] = o + b_ref[...]


@functools.partial(jax.jit, static_argnames=())
def kernel(x, adj1, adj2, W, b):
    n, two_dim = x.shape
    dim = two_dim // 2
    out_f = W.shape[0]

    x1 = x[:, :dim].astype(jnp.bfloat16)
    x2 = x[:, dim:].astype(jnp.bfloat16)
    wt = W.T
    w1 = wt[:dim, :]
    w2 = wt[dim:, :]
    b2 = b.reshape(1, out_f)

    bm = 200 if n % 200 == 0 else (8 if n % 8 == 0 else 1)
    grid = (n // bm,)

    return pl.pallas_call(
        _fused_block,
        grid=grid,
        in_specs=[
            pl.BlockSpec((bm, n), lambda i: (i, 0)),
            pl.BlockSpec((bm, n), lambda i: (i, 0)),
            pl.BlockSpec((n, dim), lambda i: (0, 0)),
            pl.BlockSpec((n, dim), lambda i: (0, 0)),
            pl.BlockSpec((dim, out_f), lambda i: (0, 0)),
            pl.BlockSpec((dim, out_f), lambda i: (0, 0)),
            pl.BlockSpec((1, out_f), lambda i: (0, 0)),
        ],
        out_specs=pl.BlockSpec((bm, out_f), lambda i: (i, 0)),
        out_shape=jax.ShapeDtypeStruct((n, out_f), jnp.float32),
        compiler_params=pltpu.CompilerParams(
            dimension_semantics=("arbitrary",),
        ),
    )(adj1, adj2, x1, x2, w1, w2, b2)
